# baseline (speedup 1.0000x reference)
import jax
import jax.numpy as jnp
from jax.experimental import pallas as pl
from jax.experimental.pallas import tpu as pltpu

N, NFEAT, NHID1, NHID2, HS = 4096, 256, 256, 128, 16

BM = 256
NB = N // BM


def _fdot(a, b):
    return jnp.dot(a, b, preferred_element_type=jnp.float32)


def _mega_kernel(x_ref, sadj_ref, fadj_ref,
                 w1_sg1_ref, w1_cg_ref, w1_sg2_ref,
                 b1_sg1_ref, b1_cg_ref, b1_sg2_ref,
                 w2_sg1_ref, w2_cg_ref, w2_sg2_ref,
                 b2s_ref, b2f_ref, attw1_ref, attb1_ref, attw2_ref,
                 beta_ref, emb1_ref, com1_ref, com2_ref, emb2_ref, emb_ref,
                 s0_s, s1_s, s2_s, ts_s, tf_s):
    p = pl.program_id(0)
    m = pl.program_id(1)

    @pl.when(jnp.logical_and(p == 0, m == 0))
    def _supports():
        xb = x_ref[...]
        s0_s[...] = _fdot(xb, w1_sg1_ref[...])
        s1_s[...] = _fdot(xb, w1_cg_ref[...])
        s2_s[...] = _fdot(xb, w1_sg2_ref[...])

    @pl.when(p == 0)
    def _layer1():
        a = sadj_ref[...]
        h_s0 = jnp.maximum(_fdot(a, s0_s[...]) + b1_sg1_ref[...], 0.0)
        h_s1 = jnp.maximum(_fdot(a, s1_s[...]) + b1_cg_ref[...], 0.0)
        ts_s[pl.ds(m * BM, BM), :] = jnp.concatenate(
            [_fdot(h_s0, w2_sg1_ref[...]),
             _fdot(h_s1, w2_cg_ref[...])], axis=1)
        f = fadj_ref[...]
        h_f1 = jnp.maximum(_fdot(f, s1_s[...]) + b1_cg_ref[...], 0.0)
        h_f2 = jnp.maximum(_fdot(f, s2_s[...]) + b1_sg2_ref[...], 0.0)
        tf_s[pl.ds(m * BM, BM), :] = jnp.concatenate(
            [_fdot(h_f1, w2_cg_ref[...]),
             _fdot(h_f2, w2_sg2_ref[...])], axis=1)

    @pl.when(p == 1)
    def _layer2_attn():
        o_s = _fdot(sadj_ref[...], ts_s[...]) + b2s_ref[...]
        o_f = _fdot(fadj_ref[...], tf_s[...]) + b2f_ref[...]
        e1 = o_s[:, :NHID2]
        c1 = o_s[:, NHID2:]
        c2 = o_f[:, :NHID2]
        e2 = o_f[:, NHID2:]
        xcom = (c1 + c2) * 0.5

        attw1 = attw1_ref[...]
        attb1 = attb1_ref[...]
        attw2 = attw2_ref[...]

        def att_logit(e):
            u = jnp.tanh(_fdot(e, attw1) + attb1)
            return jnp.sum(u * attw2, axis=1, keepdims=True)

        w0 = att_logit(e1)
        w1 = att_logit(e2)
        w2 = att_logit(xcom)
        mx = jnp.maximum(jnp.maximum(w0, w1), w2)
        p0 = jnp.exp(w0 - mx)
        p1 = jnp.exp(w1 - mx)
        p2 = jnp.exp(w2 - mx)
        denom = p0 + p1 + p2
        b0 = p0 / denom
        b1 = p1 / denom
        b2 = p2 / denom

        beta_ref[...] = jnp.concatenate([b0, b1, b2], axis=1)
        emb1_ref[...] = e1
        com1_ref[...] = c1
        com2_ref[...] = c2
        emb2_ref[...] = e2
        emb_ref[...] = b0 * e1 + b1 * e2 + b2 * xcom


def kernel(x, sadj, fadj,
           sg1_W1, sg1_b1, sg1_W2, sg1_b2,
           sg2_W1, sg2_b1, sg2_W2, sg2_b2,
           cg_W1, cg_b1, cg_W2, cg_b2,
           att_W1, att_b1, att_W2):
    f32 = jnp.float32
    bf16 = jnp.bfloat16

    b1_sg1 = sg1_b1.reshape(1, NHID1)
    b1_cg = cg_b1.reshape(1, NHID1)
    b1_sg2 = sg2_b1.reshape(1, NHID1)
    b2s = jnp.concatenate([sg1_b2, cg_b2]).reshape(1, 2 * NHID2)
    b2f = jnp.concatenate([cg_b2, sg2_b2]).reshape(1, 2 * NHID2)
    attb1 = att_b1.reshape(1, HS)
    attw2 = att_W2.reshape(1, HS)

    const = lambda r, c: pl.BlockSpec((r, c), lambda p, m: (0, 0))
    rowblk = pl.BlockSpec((BM, N), lambda p, m: (m, 0))
    outblk = lambda c: pl.BlockSpec((BM, c), lambda p, m: (p * m, 0))

    beta3, emb1, com1, com2, emb2, emb = pl.pallas_call(
        _mega_kernel,
        grid=(2, NB),
        in_specs=[
            const(N, NFEAT),
            rowblk, rowblk,
            const(NFEAT, NHID1), const(NFEAT, NHID1), const(NFEAT, NHID1),
            const(1, NHID1), const(1, NHID1), const(1, NHID1),
            const(NHID1, NHID2), const(NHID1, NHID2), const(NHID1, NHID2),
            const(1, 2 * NHID2), const(1, 2 * NHID2),
            const(NHID2, HS), const(1, HS), const(1, HS),
        ],
        out_specs=[
            outblk(3), outblk(NHID2), outblk(NHID2), outblk(NHID2),
            outblk(NHID2), outblk(NHID2),
        ],
        out_shape=[
            jax.ShapeDtypeStruct((N, 3), f32),
            jax.ShapeDtypeStruct((N, NHID2), f32),
            jax.ShapeDtypeStruct((N, NHID2), f32),
            jax.ShapeDtypeStruct((N, NHID2), f32),
            jax.ShapeDtypeStruct((N, NHID2), f32),
            jax.ShapeDtypeStruct((N, NHID2), f32),
        ],
        scratch_shapes=[
            pltpu.VMEM((N, NHID1), f32),
            pltpu.VMEM((N, NHID1), f32),
            pltpu.VMEM((N, NHID1), f32),
            pltpu.VMEM((N, 2 * NHID2), f32),
            pltpu.VMEM((N, 2 * NHID2), f32),
        ],
    )(x, sadj, fadj,
      sg1_W1, cg_W1, sg2_W1,
      b1_sg1, b1_cg, b1_sg2,
      sg1_W2, cg_W2, sg2_W2,
      b2s, b2f, att_W1, attb1, attw2)

    beta = beta3.reshape(N, 3, 1)
    return (beta, emb1, com1, com2, emb2, emb)


# speedup vs baseline: 1.0575x; 1.0575x over previous
import jax
import jax.numpy as jnp
from jax.experimental import pallas as pl
from jax.experimental.pallas import tpu as pltpu

N, NFEAT, NHID1, NHID2, HS = 4096, 256, 256, 128, 16

BM = 512
NB = N // BM


def _fdot(a, b):
    return jnp.dot(a, b, preferred_element_type=jnp.float32)


def _bdot(a, b):
    return jnp.dot(a.astype(jnp.bfloat16), b,
                   preferred_element_type=jnp.float32)


def _mega_kernel(x_ref, sadj_ref, fadj_ref,
                 w1_sg1_ref, w1_cg_ref, w1_sg2_ref,
                 b1_sg1_ref, b1_cg_ref, b1_sg2_ref,
                 w2_sg1_ref, w2_cg_ref, w2_sg2_ref,
                 b2s_ref, b2f_ref, attw1_ref, attb1_ref, attw2_ref,
                 beta_ref, emb1_ref, com1_ref, com2_ref, emb2_ref, emb_ref,
                 s0_s, s1_s, s2_s, ts_s, tf_s):
    p = pl.program_id(0)
    m = pl.program_id(1)

    @pl.when(jnp.logical_and(p == 0, m == 0))
    def _supports():
        xb = x_ref[...]
        s0_s[...] = _fdot(xb, w1_sg1_ref[...]).astype(jnp.bfloat16)
        s1_s[...] = _fdot(xb, w1_cg_ref[...]).astype(jnp.bfloat16)
        s2_s[...] = _fdot(xb, w1_sg2_ref[...]).astype(jnp.bfloat16)

    @pl.when(p == 0)
    def _layer1():
        a = sadj_ref[...].astype(jnp.bfloat16)
        h_s0 = jnp.maximum(
            jnp.dot(a, s0_s[...], preferred_element_type=jnp.float32)
            + b1_sg1_ref[...], 0.0)
        h_s1 = jnp.maximum(
            jnp.dot(a, s1_s[...], preferred_element_type=jnp.float32)
            + b1_cg_ref[...], 0.0)
        ts_s[pl.ds(m * BM, BM), :] = jnp.concatenate(
            [_bdot(h_s0, w2_sg1_ref[...].astype(jnp.bfloat16)),
             _bdot(h_s1, w2_cg_ref[...].astype(jnp.bfloat16))],
            axis=1).astype(jnp.bfloat16)
        f = fadj_ref[...].astype(jnp.bfloat16)
        h_f1 = jnp.maximum(
            jnp.dot(f, s1_s[...], preferred_element_type=jnp.float32)
            + b1_cg_ref[...], 0.0)
        h_f2 = jnp.maximum(
            jnp.dot(f, s2_s[...], preferred_element_type=jnp.float32)
            + b1_sg2_ref[...], 0.0)
        tf_s[pl.ds(m * BM, BM), :] = jnp.concatenate(
            [_bdot(h_f1, w2_cg_ref[...].astype(jnp.bfloat16)),
             _bdot(h_f2, w2_sg2_ref[...].astype(jnp.bfloat16))],
            axis=1).astype(jnp.bfloat16)

    @pl.when(p == 1)
    def _layer2_attn():
        o_s = (jnp.dot(sadj_ref[...].astype(jnp.bfloat16), ts_s[...],
                       preferred_element_type=jnp.float32) + b2s_ref[...])
        o_f = (jnp.dot(fadj_ref[...].astype(jnp.bfloat16), tf_s[...],
                       preferred_element_type=jnp.float32) + b2f_ref[...])
        e1 = o_s[:, :NHID2]
        c1 = o_s[:, NHID2:]
        c2 = o_f[:, :NHID2]
        e2 = o_f[:, NHID2:]
        xcom = (c1 + c2) * 0.5

        attw1 = attw1_ref[...]
        attb1 = attb1_ref[...]
        attw2 = attw2_ref[...]

        def att_logit(e):
            u = jnp.tanh(_fdot(e, attw1) + attb1)
            return jnp.sum(u * attw2, axis=1, keepdims=True)

        w0 = att_logit(e1)
        w1 = att_logit(e2)
        w2 = att_logit(xcom)
        mx = jnp.maximum(jnp.maximum(w0, w1), w2)
        p0 = jnp.exp(w0 - mx)
        p1 = jnp.exp(w1 - mx)
        p2 = jnp.exp(w2 - mx)
        denom = p0 + p1 + p2
        b0 = p0 / denom
        b1 = p1 / denom
        b2 = p2 / denom

        beta_ref[...] = jnp.concatenate([b0, b1, b2], axis=1)
        emb1_ref[...] = e1
        com1_ref[...] = c1
        com2_ref[...] = c2
        emb2_ref[...] = e2
        emb_ref[...] = b0 * e1 + b1 * e2 + b2 * xcom


def kernel(x, sadj, fadj,
           sg1_W1, sg1_b1, sg1_W2, sg1_b2,
           sg2_W1, sg2_b1, sg2_W2, sg2_b2,
           cg_W1, cg_b1, cg_W2, cg_b2,
           att_W1, att_b1, att_W2):
    f32 = jnp.float32
    bf16 = jnp.bfloat16

    b1_sg1 = sg1_b1.reshape(1, NHID1)
    b1_cg = cg_b1.reshape(1, NHID1)
    b1_sg2 = sg2_b1.reshape(1, NHID1)
    b2s = jnp.concatenate([sg1_b2, cg_b2]).reshape(1, 2 * NHID2)
    b2f = jnp.concatenate([cg_b2, sg2_b2]).reshape(1, 2 * NHID2)
    attb1 = att_b1.reshape(1, HS)
    attw2 = att_W2.reshape(1, HS)

    const = lambda r, c: pl.BlockSpec((r, c), lambda p, m: (0, 0))
    rowblk = pl.BlockSpec((BM, N), lambda p, m: (m, 0))
    outblk = lambda c: pl.BlockSpec((BM, c), lambda p, m: (p * m, 0))

    beta3, emb1, com1, com2, emb2, emb = pl.pallas_call(
        _mega_kernel,
        grid=(2, NB),
        in_specs=[
            const(N, NFEAT),
            rowblk, rowblk,
            const(NFEAT, NHID1), const(NFEAT, NHID1), const(NFEAT, NHID1),
            const(1, NHID1), const(1, NHID1), const(1, NHID1),
            const(NHID1, NHID2), const(NHID1, NHID2), const(NHID1, NHID2),
            const(1, 2 * NHID2), const(1, 2 * NHID2),
            const(NHID2, HS), const(1, HS), const(1, HS),
        ],
        out_specs=[
            outblk(3), outblk(NHID2), outblk(NHID2), outblk(NHID2),
            outblk(NHID2), outblk(NHID2),
        ],
        out_shape=[
            jax.ShapeDtypeStruct((N, 3), f32),
            jax.ShapeDtypeStruct((N, NHID2), f32),
            jax.ShapeDtypeStruct((N, NHID2), f32),
            jax.ShapeDtypeStruct((N, NHID2), f32),
            jax.ShapeDtypeStruct((N, NHID2), f32),
            jax.ShapeDtypeStruct((N, NHID2), f32),
        ],
        scratch_shapes=[
            pltpu.VMEM((N, NHID1), bf16),
            pltpu.VMEM((N, NHID1), bf16),
            pltpu.VMEM((N, NHID1), bf16),
            pltpu.VMEM((N, 2 * NHID2), bf16),
            pltpu.VMEM((N, 2 * NHID2), bf16),
        ],
    )(x, sadj, fadj,
      sg1_W1, cg_W1, sg2_W1,
      b1_sg1, b1_cg, b1_sg2,
      sg1_W2, cg_W2, sg2_W2,
      b2s, b2f, att_W1, attb1, attw2)

    beta = beta3.reshape(N, 3, 1)
    return (beta, emb1, com1, com2, emb2, emb)
